# Initial kernel scaffold; baseline (speedup 1.0000x reference)
import jax
import jax.numpy as jnp
from jax.experimental import pallas as pl


def kernel(x, weights, indices, counts, W_gate, W_up, W_down):
    raise NotImplementedError("write your pallas kernel here")


# speedup vs baseline: 1.8815x; 1.8815x over previous
import functools

import jax
import jax.numpy as jnp
from jax import lax
from jax.experimental import pallas as pl
from jax.experimental.pallas import tpu as pltpu
from jax.experimental.pallas import tpu_sc as plsc

_T, _D, _H, _E, _K = 2048, 2048, 1024, 8, 2
_N = _T * _K
_B = 256
_NB = _N // _B + _E
_NP = _NB * _B
_NW = 32
_TPW = _T // _NW
_CH = 32
_TCH = 16


def _route_body(idx_ref, cnt_ref, meta_ref, pos_ref):
    i32, f32 = jnp.int32, jnp.float32
    e_arr = idx_ref[...]
    cnt = cnt_ref[...]
    nbv = (cnt + (_B - 1)) >> 8
    pv_f = (nbv << 8).astype(f32)
    nbv_f = nbv.astype(f32)

    offs = []
    acc = jnp.zeros((1, 1), f32)
    for e in range(_E):
        offs.append(acc)
        acc = acc + pv_f[:, e:e + 1]
    cums = []
    cacc = jnp.zeros((1, 1), f32)
    for e in range(_E):
        cacc = cacc + nbv_f[:, e:e + 1]
        cums.append(cacc)

    U = (lax.broadcasted_iota(i32, (128, 128), 0)
         < lax.broadcasted_iota(i32, (128, 128), 1)).astype(f32)
    A = (lax.broadcasted_iota(i32, (32, 32), 1)
         < lax.broadcasted_iota(i32, (32, 32), 0)).astype(f32)
    pos_f = jnp.zeros((32, 128), f32)
    for e in range(_E):
        m = (e_arr == e).astype(f32)
        rank_row = jnp.dot(m, U, preferred_element_type=f32)
        rtot = jnp.sum(m, axis=1, keepdims=True)
        roff = jnp.dot(A, rtot, preferred_element_type=f32)
        pos_f = pos_f + m * (rank_row + roff + offs[e])
    pos_ref[...] = pos_f.astype(i32)

    b_iota = lax.broadcasted_iota(i32, (1, 128), 1)
    be = jnp.zeros((1, 128), i32)
    for e in range(_E):
        be = be + (b_iota >= cums[e].astype(i32)).astype(i32)
    be = jnp.minimum(be, _E - 1)
    nbu = cums[_E - 1].astype(i32)
    meta_ref[...] = jnp.where(b_iota < _NB, be, nbu)


_route = pl.pallas_call(
    _route_body,
    out_shape=(
        jax.ShapeDtypeStruct((1, 128), jnp.int32),
        jax.ShapeDtypeStruct((32, 128), jnp.int32),
    ),
)


def _gmm_body(be_ref, nbu_ref, xs_ref, wg_ref, wu_ref, wd_ref, ys_ref):
    del be_ref
    b = pl.program_id(0)

    @pl.when(b < nbu_ref[0])
    def _():
        bf16, f32 = jnp.bfloat16, jnp.float32
        xb = xs_ref[...].astype(bf16)
        g = jnp.dot(xb, wg_ref[0].astype(bf16), preferred_element_type=f32)
        u = jnp.dot(xb, wu_ref[0].astype(bf16), preferred_element_type=f32)
        hb = (g / (1.0 + jnp.exp(-g)) * u).astype(bf16)
        ys_ref[...] = jnp.dot(hb, wd_ref[0].astype(bf16),
                              preferred_element_type=f32)


_gmm = pl.pallas_call(
    _gmm_body,
    grid_spec=pltpu.PrefetchScalarGridSpec(
        num_scalar_prefetch=2,
        grid=(_NB,),
        in_specs=[
            pl.BlockSpec((_B, _D), lambda b, be, nbu: (b, 0)),
            pl.BlockSpec((1, _D, _H), lambda b, be, nbu: (be[b], 0, 0)),
            pl.BlockSpec((1, _D, _H), lambda b, be, nbu: (be[b], 0, 0)),
            pl.BlockSpec((1, _H, _D), lambda b, be, nbu: (be[b], 0, 0)),
        ],
        out_specs=pl.BlockSpec((_B, _D), lambda b, be, nbu: (b, 0)),
    ),
    out_shape=jax.ShapeDtypeStruct((_NP, _D), jnp.float32),
    compiler_params=pltpu.CompilerParams(
        dimension_semantics=("arbitrary",),
        vmem_limit_bytes=120 * 1024 * 1024,
    ),
)


def _sc_mesh():
    return plsc.VectorSubcoreMesh(core_axis_name="c", subcore_axis_name="s")


def _dispatch_body(x_hbm, pe_hbm, po_hbm, xs_hbm, rows_v, pe_v, po_v, sem):
    wid = lax.axis_index("s") * 2 + lax.axis_index("c")
    tbase = wid * _TPW
    for c in range(_TPW // _CH):
        tb = tbase + c * _CH
        pltpu.sync_copy(x_hbm.at[pl.ds(tb, _CH)], rows_v)
        pltpu.sync_copy(pe_hbm.at[pl.ds(tb, _CH)], pe_v)
        pltpu.sync_copy(po_hbm.at[pl.ds(tb, _CH)], po_v)
        cp1 = pltpu.async_copy(rows_v, xs_hbm.at[pe_v], sem)
        cp2 = pltpu.async_copy(rows_v, xs_hbm.at[po_v], sem)
        cp1.wait()
        cp2.wait()


def _combine_body(ys_hbm, pos_hbm, w_hbm, y_hbm, pidx_v, w_v, rows_v, out_v,
                  sem):
    wid = lax.axis_index("s") * 2 + lax.axis_index("c")
    tbase = wid * _TPW
    for c in range(_TPW // _TCH):
        tb = tbase + c * _TCH
        pltpu.sync_copy(pos_hbm.at[pl.ds(2 * tb, 2 * _TCH)], pidx_v)
        pltpu.sync_copy(w_hbm.at[pl.ds(2 * tb, 2 * _TCH)], w_v)
        pltpu.async_copy(ys_hbm.at[pidx_v], rows_v, sem).wait()
        for tt in range(_TCH):
            wv = w_v[pl.ds((tt // 8) * 16, 16)]
            w0 = jnp.full((16,), wv[(2 * tt) % 16], jnp.float32)
            w1 = jnp.full((16,), wv[(2 * tt) % 16 + 1], jnp.float32)

            def body(dd, carry, tt=tt, w0=w0, w1=w1):
                base = dd * 128
                for u in range(8):
                    sl = pl.ds(base + u * 16, 16)
                    out_v[tt, sl] = (w0 * rows_v[2 * tt, sl]
                                     + w1 * rows_v[2 * tt + 1, sl])
                return carry

            lax.fori_loop(0, _D // 128, body, 0)
        pltpu.sync_copy(out_v, y_hbm.at[pl.ds(tb, _TCH)])


@functools.cache
def _sc_kernels():
    mesh = _sc_mesh()
    dispatch = pl.kernel(
        _dispatch_body,
        out_type=jax.ShapeDtypeStruct((_NP, _D), jnp.float32),
        mesh=mesh,
        scratch_types=[
            pltpu.VMEM((_CH, _D), jnp.float32),
            pltpu.VMEM((_CH,), jnp.int32),
            pltpu.VMEM((_CH,), jnp.int32),
            pltpu.SemaphoreType.DMA,
        ],
    )
    combine = pl.kernel(
        _combine_body,
        out_type=jax.ShapeDtypeStruct((_T, _D), jnp.float32),
        mesh=mesh,
        scratch_types=[
            pltpu.VMEM((2 * _TCH,), jnp.int32),
            pltpu.VMEM((2 * _TCH,), jnp.float32),
            pltpu.VMEM((2 * _TCH, _D), jnp.float32),
            pltpu.VMEM((_TCH, _D), jnp.float32),
            pltpu.SemaphoreType.DMA,
        ],
    )
    return dispatch, combine


def kernel(x, weights, indices, counts, W_gate, W_up, W_down):
    _dispatch, _combine = _sc_kernels()
    idx2d = indices.astype(jnp.int32).reshape(32, 128)
    cnt2d = counts.astype(jnp.int32).reshape(1, _E)
    meta, pos2d = _route(idx2d, cnt2d)
    pos = pos2d.reshape(_N)
    be = meta[0, :_NB]
    nbu = meta[0, _NB:_NB + 1]
    posTK = pos2d.reshape(_T, _K)
    xs = _dispatch(x, posTK[:, 0], posTK[:, 1])
    ys = _gmm(be, nbu, xs, W_gate, W_up, W_down)
    return _combine(ys, pos, weights.reshape(_N))
